# Initial kernel scaffold; baseline (speedup 1.0000x reference)
import jax
import jax.numpy as jnp
from jax.experimental import pallas as pl


def kernel(x, edge_index0, edge_index1, edge_index2, W1, b1, W2, b2):
    raise NotImplementedError("write your pallas kernel here")


# speedup vs baseline: 4.0467x; 4.0467x over previous
import functools

import jax
import jax.numpy as jnp
from jax import lax
from jax.experimental import pallas as pl
from jax.experimental.pallas import tpu as pltpu
from jax.experimental.pallas import tpu_sc as plsc

N = 10000
E = 160000
F = 256
FH = 128
NC = 2
NS = 16
NW = NC * NS
NPAD = 10016
ACC_ROWS = 10240

_mesh = plsc.VectorSubcoreMesh(
    core_axis_name="c", subcore_axis_name="s", num_cores=NC, num_subcores=NS)


_DEG_CH = 640
_DEG_NCH = 6 * E // _DEG_CH
_DEG_PER_ARR = E // _DEG_CH
_DEG_ITERS = (_DEG_NCH + NW - 1) // NW


@functools.partial(
    pl.kernel,
    out_type=jax.ShapeDtypeStruct((NW, 6 * NPAD), jnp.float32),
    mesh=_mesh,
    compiler_params=pltpu.CompilerParams(needs_layout_passes=False, use_tc_tiling_on_sc=False),
    scratch_types=[
        pltpu.VMEM((6 * NPAD,), jnp.float32),
        pltpu.VMEM((_DEG_CH,), jnp.int32),
    ],
)
def _deg_kernel(edges, out, hist, idxbuf):
    cid = lax.axis_index("c")
    sid = lax.axis_index("s")
    wid = sid * NC + cid

    zeros16 = jnp.zeros((16,), jnp.float32)

    def zero_body(k, _):
        hist[pl.ds(k * 16, 16)] = zeros16
        return _
    lax.fori_loop(0, 6 * NPAD // 16, zero_body, None)

    ones16 = jnp.ones((16,), jnp.float32)

    def chunk_body(r, _):
        c = wid + r * NW

        @pl.when(c < _DEG_NCH)
        def _():
            a = c // _DEG_PER_ARR
            off = a * NPAD
            pltpu.sync_copy(edges.at[pl.ds(c * _DEG_CH, _DEG_CH)], idxbuf)

            def scat_body(k, _):
                v = idxbuf[pl.ds(k * 16, 16)] + off
                plsc.addupdate_scatter(hist, [v], ones16)
                return _
            lax.fori_loop(0, _DEG_CH // 16, scat_body, None)
        return _
    lax.fori_loop(0, _DEG_ITERS, chunk_body, None)

    pltpu.sync_copy(hist, out.at[wid])


_CH = 80
_EPT = E // NS
_NCH = _EPT // _CH
_ZROWS = ACC_ROWS // NS
_WROWS = N // NS


@functools.partial(
    pl.kernel,
    out_type=jax.ShapeDtypeStruct((3 * 2 * N, FH), jnp.float32),
    mesh=_mesh,
    compiler_params=pltpu.CompilerParams(needs_layout_passes=False, use_tc_tiling_on_sc=False),
    scratch_types=[
        pltpu.VMEM_SHARED((ACC_ROWS, FH), jnp.float32),
        pltpu.VMEM((64, FH), jnp.float32),
        pltpu.VMEM((_CH,), jnp.int32),
        pltpu.VMEM((_CH,), jnp.int32),
        pltpu.VMEM((_CH,), jnp.int32),
        pltpu.VMEM((_CH,), jnp.int32),
        pltpu.VMEM((_CH, FH), jnp.float32),
        pltpu.VMEM((_CH, FH), jnp.float32),
        pltpu.SemaphoreType.DMA,
        pltpu.SemaphoreType.DMA,
    ],
)
def _spmm_kernel(hsrc, edges, agg, acc, zbuf, sidx0, sidx1, didx0, didx1,
                 rows0, rows1, sem0, sem1):
    cid = lax.axis_index("c")
    sid = lax.axis_index("s")
    sidxs = (sidx0, sidx1)
    didxs = (didx0, didx1)
    rows = (rows0, rows1)
    sems = (sem0, sem1)

    zeros16 = jnp.zeros((16,), jnp.float32)

    def zfill(k, _):
        for r in range(64):
            zbuf[r, pl.ds(k * 16, 16)] = zeros16
        return _
    lax.fori_loop(0, FH // 16, zfill, None)

    for s in range(3):
        row_off = s * 2 * N + cid * N
        sbase = 2 * s * E + sid * _EPT
        dbase = (2 * s + 1) * E + sid * _EPT

        def zero_acc(r, _):
            pltpu.sync_copy(zbuf, acc.at[pl.ds(sid * _ZROWS + r * 64, 64)])
            return _
        lax.fori_loop(0, _ZROWS // 64, zero_acc, None)
        plsc.subcore_barrier()

        def load_and_fire(i, b):
            sb, db, rb, smb = sidxs[b], didxs[b], rows[b], sems[b]
            pltpu.sync_copy(edges.at[pl.ds(sbase + i * _CH, _CH)], sb)
            for j in range(_CH // 16):
                v = sb[pl.ds(j * 16, 16)]
                sb[pl.ds(j * 16, 16)] = v + row_off
            pltpu.sync_copy(edges.at[pl.ds(dbase + i * _CH, _CH)], db)
            pltpu.async_copy(hsrc.at[sb], rb, smb)

        def drain_and_scatter(b):
            sb, db, rb, smb = sidxs[b], didxs[b], rows[b], sems[b]
            pltpu.make_async_copy(hsrc.at[sb], rb, smb).wait()
            pltpu.sync_copy(rb, acc.at[db], add=True)

        load_and_fire(0, 0)
        load_and_fire(1, 1)

        def pair_body(k, _):
            for b in range(2):
                i = 2 * k + b
                drain_and_scatter(b)

                @pl.when(i + 2 < _NCH)
                def _():
                    load_and_fire(i + 2, b)
            return _
        lax.fori_loop(0, _NCH // 2, pair_body, None)
        drain_and_scatter(0)

        plsc.subcore_barrier()
        pltpu.sync_copy(
            acc.at[pl.ds(sid * _WROWS, _WROWS)],
            agg.at[pl.ds(row_off + sid * _WROWS, _WROWS)])
        plsc.subcore_barrier()


_B = 1000
_NB = N // _B


def _norms_body(hp_ref, out_ref):
    deg = jnp.sum(hp_ref[...], axis=0)
    norm = jnp.where(deg > 0, lax.rsqrt(deg), 0.0)
    out_ref[...] = norm[:, :N].reshape(3, 2, N)


def _h1_body(x_ref, w1_ref, nrm_ref, out_ref):
    xw = jnp.dot(x_ref[...], w1_ref[...], preferred_element_type=jnp.float32)
    for s in range(3):
        h = xw * nrm_ref[s, :, 0][:, None]
        out_ref[s, 0] = h[:, :FH]
        out_ref[s, 1] = h[:, FH:]


def _post1_body(agg_ref, nrm_ref, w2_ref, b1_ref, out_ref):
    aggv = jnp.concatenate([agg_ref[0, 0], agg_ref[0, 1]], axis=1)
    nin = nrm_ref[0, :, 1][:, None]
    nout = nrm_ref[0, :, 0][:, None]
    h = jnp.maximum(aggv * nin + b1_ref[...], 0.0) * nout
    hw = jnp.dot(h, w2_ref[...], preferred_element_type=jnp.float32)
    out_ref[0, 0] = hw[:, :FH]
    out_ref[0, 1] = hw[:, FH:]


def _post2_body(agg_ref, nrm_ref, b2_ref, out_ref):
    aggv = jnp.concatenate([agg_ref[0, 0], agg_ref[0, 1]], axis=1)
    nin = nrm_ref[0, :, 1][:, None]
    out_ref[0] = aggv * nin + b2_ref[...]


_norms_call = pl.pallas_call(
    _norms_body,
    out_shape=jax.ShapeDtypeStruct((3, 2, N), jnp.float32),
)

_h1_call = pl.pallas_call(
    _h1_body,
    grid=(_NB,),
    in_specs=[
        pl.BlockSpec((_B, F), lambda i: (i, 0)),
        pl.BlockSpec((F, F), lambda i: (0, 0)),
        pl.BlockSpec((3, _B, 2), lambda i: (0, i, 0)),
    ],
    out_specs=pl.BlockSpec((3, 2, _B, FH), lambda i: (0, 0, i, 0)),
    out_shape=jax.ShapeDtypeStruct((3, 2, N, FH), jnp.float32),
)

_post1_call = pl.pallas_call(
    _post1_body,
    grid=(3, _NB),
    in_specs=[
        pl.BlockSpec((1, 2, _B, FH), lambda s, i: (s, 0, i, 0)),
        pl.BlockSpec((1, _B, 2), lambda s, i: (s, i, 0)),
        pl.BlockSpec((F, F), lambda s, i: (0, 0)),
        pl.BlockSpec((F,), lambda s, i: (0,)),
    ],
    out_specs=pl.BlockSpec((1, 2, _B, FH), lambda s, i: (s, 0, i, 0)),
    out_shape=jax.ShapeDtypeStruct((3, 2, N, FH), jnp.float32),
)

_post2_call = pl.pallas_call(
    _post2_body,
    grid=(3, _NB),
    in_specs=[
        pl.BlockSpec((1, 2, _B, FH), lambda s, i: (s, 0, i, 0)),
        pl.BlockSpec((1, _B, 2), lambda s, i: (s, i, 0)),
        pl.BlockSpec((F,), lambda s, i: (0,)),
    ],
    out_specs=pl.BlockSpec((1, _B, F), lambda s, i: (s, i, 0)),
    out_shape=jax.ShapeDtypeStruct((3, N, F), jnp.float32),
)


def kernel(x, edge_index0, edge_index1, edge_index2, W1, b1, W2, b2):
    edges = jnp.concatenate([
        edge_index0.astype(jnp.int32),
        edge_index1.astype(jnp.int32),
        edge_index2.astype(jnp.int32),
    ], axis=0).reshape(-1)

    hp = _deg_kernel(edges).reshape(NW, 6, NPAD)
    norms = _norms_call(hp).transpose(0, 2, 1)
    h1 = _h1_call(x, W1, norms)
    agg1 = _spmm_kernel(h1.reshape(6 * N, FH), edges)
    h2 = _post1_call(agg1.reshape(3, 2, N, FH), norms, W2, b1)
    agg2 = _spmm_kernel(h2.reshape(6 * N, FH), edges)
    return _post2_call(agg2.reshape(3, 2, N, FH), norms, b2)
